# baseline (speedup 1.0000x reference)
import functools

import jax
import jax.numpy as jnp
from jax import lax
from jax.experimental import pallas as pl
from jax.experimental.pallas import tpu as pltpu
from jax.experimental.pallas import tpu_sc as plsc

_EPS = 1e-4
_NW = 32
_ROW_BLK = 512


def _rpad(n):
    return ((n + 1 + _ROW_BLK - 1) // _ROW_BLK) * _ROW_BLK


def _cdiv(a, b):
    return -(-a // b)


def _bn_stats(x, g, b, n_true):
    _, c = x.shape

    def body(x_ref, g_ref, b_ref, sc_ref, sh_ref):
        xx = x_ref[...]
        s = jnp.sum(xx, axis=0, keepdims=True)
        s2 = jnp.sum(xx * xx, axis=0, keepdims=True)
        mu = s / n_true
        var = s2 / n_true - mu * mu
        sc = g_ref[...] * lax.rsqrt(var + _EPS)
        sc_ref[...] = sc
        sh_ref[...] = b_ref[...] - mu * sc

    return pl.pallas_call(
        body,
        out_shape=(jax.ShapeDtypeStruct((1, c), jnp.float32),
                   jax.ShapeDtypeStruct((1, c), jnp.float32)),
    )(x, g.reshape(1, c), b.reshape(1, c))


def _matmul_bn(x, scale, shift, w_cat, n_true, apply_bn):
    r, cin = x.shape
    kc = w_cat.shape[1]
    grid = r // _ROW_BLK

    def body(x_ref, s_ref, t_ref, w_ref, y_ref):
        z = x_ref[...]
        if apply_bn:
            z = jnp.maximum(z * s_ref[...] + t_ref[...], 0.0)
        rows = (pl.program_id(0) * _ROW_BLK
                + lax.broadcasted_iota(jnp.int32, (_ROW_BLK, 1), 0))
        z = jnp.where(rows < n_true, z, 0.0)
        y_ref[...] = jnp.dot(z, w_ref[...], preferred_element_type=jnp.float32,
                             precision=lax.Precision.HIGHEST)

    return pl.pallas_call(
        body,
        grid=(grid,),
        in_specs=[
            pl.BlockSpec((_ROW_BLK, cin), lambda i: (i, 0)),
            pl.BlockSpec((1, cin), lambda i: (0, 0)),
            pl.BlockSpec((1, cin), lambda i: (0, 0)),
            pl.BlockSpec((cin, kc), lambda i: (0, 0)),
        ],
        out_specs=pl.BlockSpec((_ROW_BLK, kc), lambda i: (i, 0)),
        out_shape=jax.ShapeDtypeStruct((r, kc), jnp.float32),
    )(x, scale, shift, w_cat)


def _bn_apply(x, scale, shift):
    def body(x_ref, s_ref, t_ref, y_ref):
        y_ref[...] = jnp.maximum(x_ref[...] * s_ref[...] + t_ref[...], 0.0)

    return pl.pallas_call(
        body, out_shape=jax.ShapeDtypeStruct(x.shape, jnp.float32),
    )(x, scale, shift)


def _gather_sum(yflat, idx2, n_groups, kk, gg, gkp, cout, r_dst):
    ngw = _cdiv(n_groups, _NW)
    mesh = plsc.VectorSubcoreMesh(
        core_axis_name="c", subcore_axis_name="s", num_cores=2, num_subcores=16)

    @functools.partial(
        pl.kernel, mesh=mesh,
        compiler_params=pltpu.CompilerParams(use_tc_tiling_on_sc=False),
        out_type=jax.ShapeDtypeStruct((r_dst, cout), jnp.float32),
        scratch_types=[
            pltpu.VMEM((ngw, gkp), jnp.int32),
            pltpu.VMEM((gkp, cout), jnp.float32),
            pltpu.VMEM((gg, cout), jnp.float32),
            pltpu.SemaphoreType.DMA,
        ],
    )
    def k(y_hbm, idx_hbm, out_hbm, idxv, buf, stage, sem):
        wid = lax.axis_index("s") * 2 + lax.axis_index("c")
        g0 = wid * ngw
        cnt = jnp.clip(n_groups - g0, 0, ngw)
        pltpu.sync_copy(idx_hbm.at[pl.ds(g0, ngw)], idxv)

        def body(g, carry):
            pltpu.async_copy(y_hbm.at[idxv.at[g]], buf, sem).wait()
            if kk == 1:
                pltpu.sync_copy(buf, out_hbm.at[pl.ds((g0 + g) * gg, gg)])
            else:
                for i in range(gg):
                    for c in range(cout // 16):
                        sl = pl.ds(c * 16, 16)
                        acc = buf[i * kk, sl]
                        for t in range(1, kk):
                            acc = acc + buf[i * kk + t, sl]
                        stage[i, sl] = acc
                pltpu.sync_copy(stage, out_hbm.at[pl.ds((g0 + g) * gg, gg)])
            return carry

        lax.fori_loop(0, cnt, body, 0)

    return k(yflat, idx2)


def _prep_idx(idx, gg, gkp, zero_idx, r_dst):
    n_dst, kg = idx.shape
    n_groups = r_dst // gg
    ngw = _cdiv(n_groups, _NW)
    full = jnp.full((_NW * ngw * gg, kg), zero_idx, jnp.int32)
    full = full.at[:n_dst].set(idx.astype(jnp.int32))
    full = full.reshape(_NW * ngw, gg * kg)
    if gkp > gg * kg:
        full = jnp.pad(full, ((0, 0), (0, gkp - gg * kg)),
                       constant_values=zero_idx)
    return full, n_groups


def _sparse_conv(x, n_src, w, tap_idx, n_dst, r_dst, kk, gg, gkp,
                 bn=None, n_bn=None):
    cin = w.shape[1]
    cout = w.shape[2]
    ktaps = w.shape[0]
    w_cat = jnp.transpose(w, (1, 0, 2)).reshape(cin, ktaps * cout)
    if bn is not None:
        scale, shift = _bn_stats(x, bn[0], bn[1], n_bn)
    else:
        scale = jnp.ones((1, cin), jnp.float32)
        shift = jnp.zeros((1, cin), jnp.float32)
    y = _matmul_bn(x, scale, shift, w_cat, n_src, bn is not None)
    yflat = y.reshape(-1, cout)
    zero_idx = n_src * ktaps
    idx2, n_groups = _prep_idx(tap_idx, gg, gkp, zero_idx, r_dst)
    return _gather_sum(yflat, idx2, n_groups, kk, gg, gkp, cout, r_dst)


def _subm(x, n, w, nbr, bn=None):
    r = x.shape[0]
    koff = jnp.arange(27, dtype=jnp.int32)[None, :]
    tap_idx = jnp.where(nbr >= 0, nbr * 27 + koff, n * 27)
    return _sparse_conv(x, n, w, tap_idx, n, r, 27, 4, 112, bn=bn, n_bn=n)


def _down(x, n_src, w, dnbr, n_dst, r_dst, bn):
    koff = jnp.arange(8, dtype=jnp.int32)[None, :]
    tap_idx = jnp.where(dnbr >= 0, dnbr * 8 + koff, n_src * 8)
    return _sparse_conv(x, n_src, w, tap_idx, n_dst, r_dst, 8, 16, 128,
                        bn=bn, n_bn=n_src)


def _deconv(x, n_src, w, parent, offidx, n_dst, r_dst, bn):
    tap_idx = (parent * 8 + offidx)[:, None].astype(jnp.int32)
    return _sparse_conv(x, n_src, w, tap_idx, n_dst, r_dst, 1, 128, 128,
                        bn=bn, n_bn=n_src)


def _unet_level(x, lvl, params, meta, n_levels):
    p = params["levels"][lvl]
    n = meta["nbr"][lvl].shape[0]
    x = _subm(x, n, p["W_enc"], meta["nbr"][lvl],
              bn=(p["enc_bn_g"], p["enc_bn_b"]))
    if lvl < n_levels - 1:
        n_c = meta["down"][lvl].shape[0]
        r_c = _rpad(n_c)
        y = _down(x, n, p["W_down"], meta["down"][lvl], n_c, r_c,
                  bn=(p["pre_bn_g"], p["pre_bn_b"]))
        y = _unet_level(y, lvl + 1, params, meta, n_levels)
        y = _deconv(y, n_c, p["W_deconv"], meta["parent"][lvl],
                    meta["offidx"][lvl], n, x.shape[0],
                    bn=(p["post_bn_g"], p["post_bn_b"]))
        x = jnp.concatenate([x, y], axis=1)
        x = _subm(x, n, p["W_dec"], meta["nbr"][lvl],
                  bn=(p["dec_bn_g"], p["dec_bn_b"]))
    return x


def kernel(features, params, coords, meta):
    n0 = features.shape[0]
    r0 = _rpad(n0)
    n_levels = len(meta["nbr"])

    xf = jnp.zeros((r0, 8), jnp.float32).at[:n0, :1].set(features)
    w_in = jnp.zeros((27, 8, params["W_in"].shape[2]),
                     jnp.float32).at[:, :1, :].set(params["W_in"])
    x = _subm(xf, n0, w_in, meta["nbr"][0], bn=None)

    x = _unet_level(x, 0, params, meta, n_levels)

    scale, shift = _bn_stats(x, params["bn_out_g"], params["bn_out_b"], n0)
    y = _bn_apply(x, scale, shift)
    return y[:n0]
